# baseline (speedup 1.0000x reference)
import functools

import jax
import jax.numpy as jnp
from jax import lax
from jax.experimental import pallas as pl
from jax.experimental.pallas import tpu as pltpu
from jax.experimental.pallas import tpu_sc as plsc

F32 = jnp.float32

_N = 10000
_D = 128
_E = 320000
_NC, _NS = 2, 16
_NW = _NC * _NS
_NP = 10240
_STRIPE = _NP // _NS
_CH = 80
_CPW = 128
_EPW = _CPW * _CH
_EP = _NW * _EPW
_EPC = _EP // _NC
_BS = 400
_NB = _N // _BS

_MESH = plsc.VectorSubcoreMesh(
    core_axis_name="c", subcore_axis_name="s",
    num_cores=_NC, num_subcores=_NS)


def _deg_body(dst2d, out0, out1, dstv, onesv, zb, deg_sh):
    ci = lax.axis_index("c")
    si = lax.axis_index("s")
    w = si * _NC + ci
    soff = pl.multiple_of(si * _STRIPE, 8)

    @pl.loop(0, _STRIPE // 16)
    def _zero(i):
        zb[pl.ds(pl.multiple_of(i * 16, 8), 16)] = jnp.zeros((16,), F32)

    pltpu.sync_copy(zb, deg_sh.at[pl.ds(soff, _STRIPE)])
    for j in range(_CH // 16):
        onesv[pl.ds(j * 16, 16)] = jnp.ones((16,), F32)
    pltpu.sync_copy(dst2d.at[pl.ds(pl.multiple_of(w * _CPW, 8), _CPW)], dstv)
    plsc.subcore_barrier()

    @pl.loop(0, _CPW)
    def _scat(c):
        pltpu.sync_copy(onesv, deg_sh.at[dstv.at[c]], add=True)

    plsc.subcore_barrier()

    @pl.when(ci == 0)
    def _w0():
        pltpu.sync_copy(deg_sh.at[pl.ds(soff, _STRIPE)],
                        out0.at[pl.ds(soff, _STRIPE)])

    @pl.when(ci == 1)
    def _w1():
        pltpu.sync_copy(deg_sh.at[pl.ds(soff, _STRIPE)],
                        out1.at[pl.ds(soff, _STRIPE)])


_deg_call = functools.partial(
    pl.kernel,
    out_type=(jax.ShapeDtypeStruct((_NP,), F32),
              jax.ShapeDtypeStruct((_NP,), F32)),
    mesh=_MESH,
    scratch_types=[
        pltpu.VMEM((_CPW, _CH), jnp.int32),
        pltpu.VMEM((_CH,), F32),
        pltpu.VMEM((_STRIPE,), F32),
        pltpu.VMEM_SHARED((_NP,), F32),
    ],
)(_deg_body)


_NBUF = 2
_NWB = 16
_EPT = _EPC // _NS
_CPT = _EPT // _CH


def _agg_body(idx, h, out, *scr):
    srcv, dstv = scr[0], scr[1]
    bufs = scr[2:2 + _NBUF]
    sems = scr[2 + _NBUF:2 + 2 * _NBUF]
    wsems = scr[2 + 2 * _NBUF:2 + 2 * _NBUF + _NWB]
    agg_sh = scr[2 + 2 * _NBUF + _NWB]
    ci = lax.axis_index("c")
    si = lax.axis_index("s")
    soff = pl.multiple_of(si * _STRIPE, 8)
    r0 = bufs[0]

    @pl.loop(0, _CH)
    def _zero(i):
        for j in range(_D // 16):
            r0[i, pl.ds(j * 16, 16)] = jnp.zeros((16,), F32)

    for k in range(_STRIPE // _CH):
        pltpu.sync_copy(r0, agg_sh.at[pl.ds(soff + k * _CH, _CH)])
    ebase = (si * _NC + ci) * _EPT
    pltpu.sync_copy(idx.at[pl.ds(pl.multiple_of(ebase, 8), _EPT)], srcv)
    pltpu.sync_copy(idx.at[pl.ds(pl.multiple_of(_EP + ebase, 8), _EPT)],
                    dstv)
    plsc.subcore_barrier()

    def g(c, j):
        iv = srcv.at[pl.ds(pl.multiple_of(c * _CH, 8), _CH)]
        return pltpu.make_async_copy(h.at[iv], bufs[j], sems[j])

    for j in range(_NBUF):
        g(j, j).start()

    @pl.loop(0, _CPT // _NBUF - 1)
    def _main(gi):
        c = gi * _NBUF
        for j in range(_NBUF):
            g(c + j, j).wait()
            pltpu.sync_copy(
                bufs[j],
                agg_sh.at[dstv.at[pl.ds(pl.multiple_of((c + j) * _CH, 8),
                                        _CH)]],
                add=True)
            g(c + j + _NBUF, j).start()

    for j in range(_NBUF):
        c = _CPT - _NBUF + j
        g(c, j).wait()
        pltpu.sync_copy(
            bufs[j],
            agg_sh.at[dstv.at[pl.ds(pl.multiple_of(c * _CH, 8), _CH)]],
            add=True)

    plsc.subcore_barrier()

    obase = ci * (_NP * _D)

    def w(r, j):
        row = soff + r
        return pltpu.make_async_copy(
            agg_sh.at[row], out.at[pl.ds(obase + row * _D, _D)], wsems[j])

    for j in range(_NWB):
        w(j, j).start()

    @pl.loop(0, _STRIPE // _NWB - 1)
    def _wr(gi):
        r = gi * _NWB
        for j in range(_NWB):
            w(r + j, j).wait()
            w(r + j + _NWB, j).start()

    for j in range(_NWB):
        w(_STRIPE - _NWB + j, j).wait()


_agg_call = functools.partial(
    pl.kernel,
    out_type=jax.ShapeDtypeStruct((2 * _NP * _D,), F32),
    mesh=_MESH,
    scratch_types=(
        [pltpu.VMEM((_EPT,), jnp.int32),
         pltpu.VMEM((_EPT,), jnp.int32)]
        + [pltpu.VMEM((_CH, _D), F32)] * _NBUF
        + [pltpu.SemaphoreType.DMA] * _NBUF
        + [pltpu.SemaphoreType.DMA] * _NWB
        + [pltpu.VMEM_SHARED((_NP, _D), F32)]
    ),
)(_agg_body)


def _agg(idx, h):
    s = _agg_call(idx, h).reshape(2, _NP, _D)
    return s[0], s[1]


def _tc2_body(d0_ref, d1_ref, x_ref, w0_ref, h_ref, dinv_ref):
    deg = d0_ref[...] + d1_ref[...]
    dinv = lax.rsqrt(deg + 1.0)
    h = jnp.dot(x_ref[...], w0_ref[...], preferred_element_type=F32)
    h_ref[...] = h * dinv
    dinv_ref[...] = dinv


def _tc2(d0, d1, x, W0):
    col = pl.BlockSpec((_BS, 1), lambda i: (i, 0))
    return pl.pallas_call(
        _tc2_body,
        grid=(_NB,),
        in_specs=[
            col, col,
            pl.BlockSpec((_BS, _D), lambda i: (i, 0)),
            pl.BlockSpec((_D, _D), lambda i: (0, 0)),
        ],
        out_specs=[
            pl.BlockSpec((_BS, _D), lambda i: (i, 0)),
            col,
        ],
        out_shape=[
            jax.ShapeDtypeStruct((_N, _D), F32),
            jax.ShapeDtypeStruct((_N, 1), F32),
        ],
    )(d0, d1, x, W0)


def _tc3_body(s0_ref, s1_ref, h_ref, dinv_ref, b0_ref, lng_ref, lnb_ref,
              mw1_ref, mb1_ref, mw2_ref, mb2_ref, w1_ref, out_ref):
    dinv = dinv_ref[...]
    s = s0_ref[...] + s1_ref[...]
    a = dinv * (s + h_ref[...]) + b0_ref[...]
    t = jnp.tanh(a)
    mu = jnp.mean(t, axis=1, keepdims=True)
    var = jnp.mean((t - mu) ** 2, axis=1, keepdims=True)
    ln = (t - mu) * lax.rsqrt(var + 1e-5) * lng_ref[...] + lnb_ref[...]
    u = jnp.dot(ln, mw1_ref[...], preferred_element_type=F32) + mb1_ref[...]
    p = jnp.dot(u, mw2_ref[...], preferred_element_type=F32) + mb2_ref[...]
    out_ref[...] = dinv * jnp.dot(p, w1_ref[...], preferred_element_type=F32)


def _tc3(s0, s1, h0, dinv, b0, lng, lnb, mW1, mb1, mW2, mb2, W1):
    blk = pl.BlockSpec((_BS, _D), lambda i: (i, 0))
    full = pl.BlockSpec((_D, _D), lambda i: (0, 0))
    row = pl.BlockSpec((1, _D), lambda i: (0, 0))
    col = pl.BlockSpec((_BS, 1), lambda i: (i, 0))
    return pl.pallas_call(
        _tc3_body,
        grid=(_NB,),
        in_specs=[blk, blk, blk, col,
                  row, row, row, full, row, full, row, full],
        out_specs=blk,
        out_shape=jax.ShapeDtypeStruct((_N, _D), F32),
    )(s0, s1, h0, dinv, b0, lng, lnb, mW1, mb1, mW2, mb2, W1)


def _tc4_body(s0_ref, s1_ref, h_ref, dinv_ref, b1_ref, mw1_ref, mb1_ref,
              mw2_ref, mb2_ref, emb_ref, out_ref):
    dinv = dinv_ref[...]
    s = s0_ref[...] + s1_ref[...]
    a = dinv * (s + h_ref[...]) + b1_ref[...]
    emb_ref[...] = a
    t = jnp.tanh(a)
    u = jnp.dot(t, mw1_ref[...], preferred_element_type=F32) + mb1_ref[...]
    p = jnp.dot(u, mw2_ref[...], preferred_element_type=F32) + mb2_ref[...]
    m = jnp.max(p, axis=1, keepdims=True)
    lse = jnp.log(jnp.sum(jnp.exp(p - m), axis=1, keepdims=True)) + m
    out_ref[...] = p - lse


def _tc4(s0, s1, h1, dinv, b1, mW1, mb1, mW2, mb2):
    blk = pl.BlockSpec((_BS, _D), lambda i: (i, 0))
    full = pl.BlockSpec((_D, _D), lambda i: (0, 0))
    row = pl.BlockSpec((1, _D), lambda i: (0, 0))
    col = pl.BlockSpec((_BS, 1), lambda i: (i, 0))
    return pl.pallas_call(
        _tc4_body,
        grid=(_NB,),
        in_specs=[blk, blk, blk, col, row, full, row, full, row],
        out_specs=[blk, blk],
        out_shape=[
            jax.ShapeDtypeStruct((_N, _D), F32),
            jax.ShapeDtypeStruct((_N, _D), F32),
        ],
    )(s0, s1, h1, dinv, b1, mW1, mb1, mW2, mb2)


def kernel(x, edge_index, batch, W0, b0, W1, b1, ln_g, ln_b,
           mW1, mb1, mW2, mb2):
    src = edge_index[0]
    dst = edge_index[1]
    pad = _EP - _E
    src1 = jnp.concatenate([src, jnp.zeros((pad,), jnp.int32)])
    dstp = jnp.concatenate([dst, jnp.full((pad,), _NP - 1, jnp.int32)])
    idx = jnp.concatenate([src1, dstp])
    dst2d = dstp.reshape(_EP // _CH, _CH)

    d0, d1 = _deg_call(dst2d)
    d0 = d0.reshape(_NP, 1)
    d1 = d1.reshape(_NP, 1)

    h0, dinv = _tc2(d0, d1, x, W0)
    s0a, s0b = _agg(idx, h0)
    h1 = _tc3(s0a, s0b, h0, dinv,
              b0.reshape(1, _D), ln_g.reshape(1, _D),
              ln_b.reshape(1, _D), mW1, mb1.reshape(1, _D),
              mW2, mb2.reshape(1, _D), W1)
    s1a, s1b = _agg(idx, h1)
    emb, out2 = _tc4(s1a, s1b, h1, dinv,
                     b1.reshape(1, _D), mW1, mb1.reshape(1, _D),
                     mW2, mb2.reshape(1, _D))
    return emb, out2
